# baseline (speedup 1.0000x reference)
import functools

import jax
import jax.numpy as jnp
from jax import lax
from jax.experimental import pallas as pl
from jax.experimental.pallas import tpu as pltpu
from jax.experimental.pallas import tpu_sc as plsc

VOCAB = 100000
EMB = 64
EMB_PAD = 128
BATCH = 1024
CTX = 50

NC = 2
NS = 16
BLANES = 32
NW = NC * NS
ITEMS_PER_W = BATCH // NW
ROWS_PER_W = ITEMS_PER_W * CTX
NPASS = 4
ITEMS_PER_PASS = ITEMS_PER_W // NPASS
ROWS_PER_PASS = ITEMS_PER_PASS * CTX
GCHUNK = 80
NSTREAM = ROWS_PER_PASS // GCHUNK


def _pool_sc_body(table_hbm, idx_hbm, out_hbm, idx_v, rows0, rows1, acc_v,
                  sem0, sem1):
    wid = lax.axis_index("s") * NC + lax.axis_index("c")
    base = wid * ROWS_PER_W
    pltpu.sync_copy(idx_hbm.at[pl.ds(base, ROWS_PER_W)], idx_v)

    def fire(p, buf, sem):
        for s in range(NSTREAM):
            pltpu.make_async_copy(
                table_hbm.at[idx_v.at[pl.ds(p * ROWS_PER_PASS + s * GCHUNK,
                                            GCHUNK)]],
                buf.at[pl.ds(s * GCHUNK, GCHUNK)],
                sem,
            ).start()

    def drain(p, buf, sem):
        for s in range(NSTREAM):
            pltpu.make_async_copy(
                table_hbm.at[idx_v.at[pl.ds(p * ROWS_PER_PASS + s * GCHUNK,
                                            GCHUNK)]],
                buf.at[pl.ds(s * GCHUNK, GCHUNK)],
                sem,
            ).wait()

    def accum(p, buf):
        @pl.loop(0, ITEMS_PER_PASS)
        def _(j):
            r0 = j * CTX

            def body(r, accs):
                return tuple(
                    accs[c] + buf[r0 + r, pl.ds(c * BLANES, BLANES)]
                    for c in range(EMB // BLANES)
                )

            init = tuple(
                buf[r0, pl.ds(c * BLANES, BLANES)]
                for c in range(EMB // BLANES)
            )
            accs = lax.fori_loop(1, CTX, body, init)
            it = p * ITEMS_PER_PASS + j
            for c in range(EMB // BLANES):
                acc_v[it, pl.ds(c * BLANES, BLANES)] = accs[c]

    bufs = (rows0, rows1)
    sems = (sem0, sem1)
    fire(0, rows0, sem0)
    fire(1, rows1, sem1)
    for p in range(NPASS):
        b = p % 2
        drain(p, bufs[b], sems[b])
        accum(p, bufs[b])
        if p + 2 < NPASS:
            fire(p + 2, bufs[b], sems[b])
    pltpu.sync_copy(acc_v, out_hbm.at[pl.ds(wid * ITEMS_PER_W, ITEMS_PER_W)])


@functools.cache
def _make_pool_sc():
    mesh = plsc.VectorSubcoreMesh(core_axis_name="c", subcore_axis_name="s")
    return pl.kernel(
        _pool_sc_body,
        mesh=mesh,
        out_type=jax.ShapeDtypeStruct((BATCH, EMB_PAD), jnp.bfloat16),
        scratch_types=[
            pltpu.VMEM((ROWS_PER_W,), jnp.int32),
            pltpu.VMEM((ROWS_PER_PASS, EMB_PAD), jnp.bfloat16),
            pltpu.VMEM((ROWS_PER_PASS, EMB_PAD), jnp.bfloat16),
            pltpu.VMEM((ITEMS_PER_W, EMB_PAD), jnp.bfloat16),
            pltpu.SemaphoreType.DMA,
            pltpu.SemaphoreType.DMA,
        ],
        compiler_params=pltpu.CompilerParams(use_tc_tiling_on_sc=False),
    )


T_BLK = 4096


def _repack_body(et_ref, o_ref):
    t = jnp.swapaxes(et_ref[...], 0, 1) * jnp.float32(1.0 / CTX)
    o_ref[:, :EMB] = t.astype(jnp.bfloat16)
    o_ref[:, EMB:] = jnp.zeros((T_BLK, EMB_PAD - EMB), jnp.bfloat16)


def _repack(emb_t):
    return pl.pallas_call(
        _repack_body,
        grid=((VOCAB + T_BLK - 1) // T_BLK,),
        in_specs=[pl.BlockSpec((EMB, T_BLK), lambda i: (0, i))],
        out_specs=pl.BlockSpec((T_BLK, EMB_PAD), lambda i: (i, 0)),
        out_shape=jax.ShapeDtypeStruct((VOCAB, EMB_PAD), jnp.bfloat16),
    )(emb_t)


V_BLK = 2048
_NVB = (VOCAB + V_BLK - 1) // V_BLK


def _mm_body(p_ref, wt_ref, b_ref, o_ref):
    p = p_ref[:, :EMB]
    wt = wt_ref[...].astype(jnp.bfloat16)
    acc = lax.dot_general(
        wt, p, (((0,), (1,)), ((), ())), preferred_element_type=jnp.float32
    )
    o_ref[...] = acc + b_ref[...].T


def _project(pooled, W_T, b2d):
    out_t = pl.pallas_call(
        _mm_body,
        grid=(_NVB,),
        in_specs=[
            pl.BlockSpec((BATCH, EMB_PAD), lambda i: (0, 0)),
            pl.BlockSpec((EMB, V_BLK), lambda i: (0, i)),
            pl.BlockSpec((1, V_BLK), lambda i: (0, i)),
        ],
        out_specs=pl.BlockSpec((V_BLK, BATCH), lambda i: (i, 0)),
        out_shape=jax.ShapeDtypeStruct((VOCAB, BATCH), jnp.float32),
    )(pooled, W_T, b2d)
    return out_t.T


def kernel(inputs, emb_table, W, b):
    idx_flat = inputs.reshape(-1).astype(jnp.int32)
    table_bf = _repack(emb_table.T)
    pooled = _make_pool_sc()(table_bf, idx_flat)
    return _project(pooled, W.T, b.reshape(1, VOCAB))


# speedup vs baseline: 1.3495x; 1.3495x over previous
import functools

import jax
import jax.numpy as jnp
from jax import lax
from jax.experimental import pallas as pl
from jax.experimental.pallas import tpu as pltpu
from jax.experimental.pallas import tpu_sc as plsc

VOCAB = 100000
EMB = 64
EMB_PAD = 128
BATCH = 1024
CTX = 50

NC = 2
NS = 16
LANES = 16
NW = NC * NS
ITEMS_PER_W = BATCH // NW
ROWS_PER_W = ITEMS_PER_W * CTX
NPASS = 4
ITEMS_PER_PASS = ITEMS_PER_W // NPASS
ROWS_PER_PASS = ITEMS_PER_PASS * CTX
GCHUNK = 80
NSTREAM = ROWS_PER_PASS // GCHUNK


def _pool_sc_body(table_hbm, idx_hbm, out_hbm, idx_v, rows0, rows1, acc_v,
                  sem0, sem1):
    wid = lax.axis_index("s") * NC + lax.axis_index("c")
    base = wid * ROWS_PER_W
    pltpu.sync_copy(idx_hbm.at[pl.ds(base, ROWS_PER_W)], idx_v)

    def fire(p, buf, sem):
        for s in range(NSTREAM):
            pltpu.make_async_copy(
                table_hbm.at[idx_v.at[pl.ds(p * ROWS_PER_PASS + s * GCHUNK,
                                            GCHUNK)]],
                buf.at[pl.ds(s * GCHUNK, GCHUNK)],
                sem,
            ).start()

    def drain(p, buf, sem):
        for s in range(NSTREAM):
            pltpu.make_async_copy(
                table_hbm.at[idx_v.at[pl.ds(p * ROWS_PER_PASS + s * GCHUNK,
                                            GCHUNK)]],
                buf.at[pl.ds(s * GCHUNK, GCHUNK)],
                sem,
            ).wait()

    def accum(p, buf):
        @pl.loop(0, ITEMS_PER_PASS)
        def _(j):
            r0 = j * CTX

            def body(r, accs):
                return tuple(
                    accs[c] + buf[r0 + r, pl.ds(c * LANES, LANES)]
                    for c in range(EMB // LANES)
                )

            init = tuple(
                buf[r0, pl.ds(c * LANES, LANES)] for c in range(EMB // LANES)
            )
            accs = lax.fori_loop(1, CTX, body, init)
            it = p * ITEMS_PER_PASS + j
            for c in range(EMB // LANES):
                acc_v[it, pl.ds(c * LANES, LANES)] = accs[c]

    bufs = (rows0, rows1)
    sems = (sem0, sem1)
    fire(0, rows0, sem0)
    fire(1, rows1, sem1)
    for p in range(NPASS):
        b = p % 2
        drain(p, bufs[b], sems[b])
        accum(p, bufs[b])
        if p + 2 < NPASS:
            fire(p + 2, bufs[b], sems[b])
    pltpu.sync_copy(acc_v, out_hbm.at[pl.ds(wid * ITEMS_PER_W, ITEMS_PER_W)])


@functools.cache
def _make_pool_sc():
    mesh = plsc.VectorSubcoreMesh(core_axis_name="c", subcore_axis_name="s")
    return pl.kernel(
        _pool_sc_body,
        mesh=mesh,
        out_type=jax.ShapeDtypeStruct((BATCH, EMB_PAD), jnp.float32),
        scratch_types=[
            pltpu.VMEM((ROWS_PER_W,), jnp.int32),
            pltpu.VMEM((ROWS_PER_PASS, EMB_PAD), jnp.float32),
            pltpu.VMEM((ROWS_PER_PASS, EMB_PAD), jnp.float32),
            pltpu.VMEM((ITEMS_PER_W, EMB_PAD), jnp.float32),
            pltpu.SemaphoreType.DMA,
            pltpu.SemaphoreType.DMA,
        ],
        compiler_params=pltpu.CompilerParams(use_tc_tiling_on_sc=False),
    )


T_BLK = 2048


def _repack_body(et_ref, o_ref):
    t = jnp.swapaxes(et_ref[...], 0, 1) * jnp.float32(1.0 / CTX)
    o_ref[:, :EMB] = t
    o_ref[:, EMB:] = jnp.zeros((T_BLK, EMB_PAD - EMB), jnp.float32)


def _repack(emb_t):
    return pl.pallas_call(
        _repack_body,
        grid=((VOCAB + T_BLK - 1) // T_BLK,),
        in_specs=[pl.BlockSpec((EMB, T_BLK), lambda i: (0, i))],
        out_specs=pl.BlockSpec((T_BLK, EMB_PAD), lambda i: (i, 0)),
        out_shape=jax.ShapeDtypeStruct((VOCAB, EMB_PAD), jnp.float32),
    )(emb_t)


V_BLK = 2048
_NVB = (VOCAB + V_BLK - 1) // V_BLK


def _mm_body(p_ref, wt_ref, b_ref, o_ref):
    p = p_ref[:, :EMB].astype(jnp.bfloat16)
    wt = wt_ref[...].astype(jnp.bfloat16)
    acc = lax.dot_general(
        wt, p, (((0,), (1,)), ((), ())), preferred_element_type=jnp.float32
    )
    o_ref[...] = acc + b_ref[...].T


def _project(pooled, W_T, b2d):
    out_t = pl.pallas_call(
        _mm_body,
        grid=(_NVB,),
        in_specs=[
            pl.BlockSpec((BATCH, EMB_PAD), lambda i: (0, 0)),
            pl.BlockSpec((EMB, V_BLK), lambda i: (0, i)),
            pl.BlockSpec((1, V_BLK), lambda i: (0, i)),
        ],
        out_specs=pl.BlockSpec((V_BLK, BATCH), lambda i: (i, 0)),
        out_shape=jax.ShapeDtypeStruct((VOCAB, BATCH), jnp.float32),
    )(pooled, W_T, b2d)
    return out_t.T


def kernel(inputs, emb_table, W, b):
    idx_flat = inputs.reshape(-1).astype(jnp.int32)
    table_s = _repack(emb_table.T)
    pooled = _make_pool_sc()(table_s, idx_flat)
    return _project(pooled, W.T, b.reshape(1, VOCAB))


# speedup vs baseline: 1.4298x; 1.0595x over previous
import functools

import jax
import jax.numpy as jnp
from jax import lax
from jax.experimental import pallas as pl
from jax.experimental.pallas import tpu as pltpu
from jax.experimental.pallas import tpu_sc as plsc

VOCAB = 100000
EMB = 64
EMB_PAD = 128
BATCH = 1024
CTX = 50

NC = 2
NS = 16
LANES = 16
NW = NC * NS
ITEMS_PER_W = BATCH // NW
ROWS_PER_W = ITEMS_PER_W * CTX
NPASS = 4
ITEMS_PER_PASS = ITEMS_PER_W // NPASS
ROWS_PER_PASS = ITEMS_PER_PASS * CTX
GCHUNK = 80
NSTREAM = ROWS_PER_PASS // GCHUNK


def _pool_sc_body(table_hbm, idx_hbm, out_hbm, idx_v, rows0, rows1, acc_v,
                  sem0, sem1):
    wid = lax.axis_index("s") * NC + lax.axis_index("c")
    base = wid * ROWS_PER_W
    pltpu.sync_copy(idx_hbm.at[pl.ds(base, ROWS_PER_W)], idx_v)

    def fire(p, buf, sem):
        for s in range(NSTREAM):
            pltpu.make_async_copy(
                table_hbm.at[idx_v.at[pl.ds(p * ROWS_PER_PASS + s * GCHUNK,
                                            GCHUNK)]],
                buf.at[pl.ds(s * GCHUNK, GCHUNK)],
                sem,
            ).start()

    def drain(p, buf, sem):
        for s in range(NSTREAM):
            pltpu.make_async_copy(
                table_hbm.at[idx_v.at[pl.ds(p * ROWS_PER_PASS + s * GCHUNK,
                                            GCHUNK)]],
                buf.at[pl.ds(s * GCHUNK, GCHUNK)],
                sem,
            ).wait()

    def accum(p, buf):
        @pl.loop(0, ITEMS_PER_PASS)
        def _(j):
            r0 = j * CTX

            def body(r, accs):
                return tuple(
                    accs[c] + buf[r0 + r, pl.ds(c * LANES, LANES)]
                    for c in range(EMB // LANES)
                )

            init = tuple(
                buf[r0, pl.ds(c * LANES, LANES)] for c in range(EMB // LANES)
            )
            accs = lax.fori_loop(1, CTX, body, init)
            it = p * ITEMS_PER_PASS + j
            for c in range(EMB // LANES):
                acc_v[it, pl.ds(c * LANES, LANES)] = accs[c]

    bufs = (rows0, rows1)
    sems = (sem0, sem1)
    fire(0, rows0, sem0)
    fire(1, rows1, sem1)
    for p in range(NPASS):
        b = p % 2
        drain(p, bufs[b], sems[b])
        accum(p, bufs[b])
        if p + 2 < NPASS:
            fire(p + 2, bufs[b], sems[b])
    pltpu.sync_copy(acc_v, out_hbm.at[pl.ds(wid * ITEMS_PER_W, ITEMS_PER_W)])


@functools.cache
def _make_pool_sc():
    mesh = plsc.VectorSubcoreMesh(core_axis_name="c", subcore_axis_name="s")
    return pl.kernel(
        _pool_sc_body,
        mesh=mesh,
        out_type=jax.ShapeDtypeStruct((BATCH, EMB_PAD), jnp.float32),
        scratch_types=[
            pltpu.VMEM((ROWS_PER_W,), jnp.int32),
            pltpu.VMEM((ROWS_PER_PASS, EMB_PAD), jnp.float32),
            pltpu.VMEM((ROWS_PER_PASS, EMB_PAD), jnp.float32),
            pltpu.VMEM((ITEMS_PER_W, EMB_PAD), jnp.float32),
            pltpu.SemaphoreType.DMA,
            pltpu.SemaphoreType.DMA,
        ],
        compiler_params=pltpu.CompilerParams(use_tc_tiling_on_sc=False),
    )


T_BLK = 4096


def _repack_body(et_ref, o_ref):
    t = jnp.swapaxes(et_ref[...], 0, 1) * jnp.float32(1.0 / CTX)
    o_ref[:, :EMB] = t
    o_ref[:, EMB:] = jnp.zeros((T_BLK, EMB_PAD - EMB), jnp.float32)


def _repack(emb_t):
    return pl.pallas_call(
        _repack_body,
        grid=((VOCAB + T_BLK - 1) // T_BLK,),
        in_specs=[pl.BlockSpec((EMB, T_BLK), lambda i: (0, i))],
        out_specs=pl.BlockSpec((T_BLK, EMB_PAD), lambda i: (i, 0)),
        out_shape=jax.ShapeDtypeStruct((VOCAB, EMB_PAD), jnp.float32),
    )(emb_t)


V_BLK = 2048
_NVB = (VOCAB + V_BLK - 1) // V_BLK


def _mm_body(p_ref, wt_ref, b_ref, o_ref):
    p = p_ref[:, :EMB].astype(jnp.bfloat16)
    wt = wt_ref[...].astype(jnp.bfloat16)
    acc = lax.dot_general(
        wt, p, (((0,), (1,)), ((), ())), preferred_element_type=jnp.float32
    )
    o_ref[...] = acc + b_ref[...].T


def _project(pooled, W_T, b2d):
    out_t = pl.pallas_call(
        _mm_body,
        grid=(_NVB,),
        in_specs=[
            pl.BlockSpec((BATCH, EMB_PAD), lambda i: (0, 0)),
            pl.BlockSpec((EMB, V_BLK), lambda i: (0, i)),
            pl.BlockSpec((1, V_BLK), lambda i: (0, i)),
        ],
        out_specs=pl.BlockSpec((V_BLK, BATCH), lambda i: (i, 0)),
        out_shape=jax.ShapeDtypeStruct((VOCAB, BATCH), jnp.float32),
    )(pooled, W_T, b2d)
    return out_t.T


def kernel(inputs, emb_table, W, b):
    idx_flat = inputs.reshape(-1).astype(jnp.int32)
    table_s = _repack(emb_table.T)
    pooled = _make_pool_sc()(table_s, idx_flat)
    return _project(pooled, W.T, b.reshape(1, VOCAB))


# speedup vs baseline: 1.4940x; 1.0449x over previous
import functools

import jax
import jax.numpy as jnp
from jax import lax
from jax.experimental import pallas as pl
from jax.experimental.pallas import tpu as pltpu
from jax.experimental.pallas import tpu_sc as plsc

VOCAB = 100000
EMB = 64
EMB_PAD = 128
BATCH = 1024
CTX = 50

NC = 2
NS = 16
LANES = 16
NW = NC * NS
ITEMS_PER_W = BATCH // NW
ROWS_PER_W = ITEMS_PER_W * CTX
NPASS = 4
ITEMS_PER_PASS = ITEMS_PER_W // NPASS
ROWS_PER_PASS = ITEMS_PER_PASS * CTX
GCHUNK = 80
NSTREAM = ROWS_PER_PASS // GCHUNK


def _pool_sc_body(table_hbm, idx_hbm, out_hbm, idx_v, rows0, rows1, acc_v,
                  sem0, sem1):
    wid = lax.axis_index("s") * NC + lax.axis_index("c")
    base = wid * ROWS_PER_W
    pltpu.sync_copy(idx_hbm.at[pl.ds(base, ROWS_PER_W)], idx_v)

    def fire(p, buf, sem):
        for s in range(NSTREAM):
            pltpu.make_async_copy(
                table_hbm.at[idx_v.at[pl.ds(p * ROWS_PER_PASS + s * GCHUNK,
                                            GCHUNK)]],
                buf.at[pl.ds(s * GCHUNK, GCHUNK)],
                sem,
            ).start()

    def drain(p, buf, sem):
        for s in range(NSTREAM):
            pltpu.make_async_copy(
                table_hbm.at[idx_v.at[pl.ds(p * ROWS_PER_PASS + s * GCHUNK,
                                            GCHUNK)]],
                buf.at[pl.ds(s * GCHUNK, GCHUNK)],
                sem,
            ).wait()

    def accum(p, buf):
        @pl.loop(0, ITEMS_PER_PASS)
        def _(j):
            r0 = j * CTX

            def body(r, accs):
                return tuple(
                    accs[c] + buf[r0 + r, pl.ds(c * LANES, LANES)]
                    for c in range(EMB // LANES)
                )

            init = tuple(
                buf[r0, pl.ds(c * LANES, LANES)] for c in range(EMB // LANES)
            )
            accs = lax.fori_loop(1, CTX, body, init)
            it = p * ITEMS_PER_PASS + j
            for c in range(EMB // LANES):
                acc_v[it, pl.ds(c * LANES, LANES)] = accs[c]

    bufs = (rows0, rows1)
    sems = (sem0, sem1)
    fire(0, rows0, sem0)
    fire(1, rows1, sem1)
    for p in range(NPASS):
        b = p % 2
        drain(p, bufs[b], sems[b])
        accum(p, bufs[b])
        if p + 2 < NPASS:
            fire(p + 2, bufs[b], sems[b])
    pltpu.sync_copy(acc_v, out_hbm.at[pl.ds(wid * ITEMS_PER_W, ITEMS_PER_W)])


@functools.cache
def _make_pool_sc():
    mesh = plsc.VectorSubcoreMesh(core_axis_name="c", subcore_axis_name="s")
    return pl.kernel(
        _pool_sc_body,
        mesh=mesh,
        out_type=jax.ShapeDtypeStruct((BATCH, EMB_PAD), jnp.float32),
        scratch_types=[
            pltpu.VMEM((ROWS_PER_W,), jnp.int32),
            pltpu.VMEM((ROWS_PER_PASS, EMB_PAD), jnp.float32),
            pltpu.VMEM((ROWS_PER_PASS, EMB_PAD), jnp.float32),
            pltpu.VMEM((ITEMS_PER_W, EMB_PAD), jnp.float32),
            pltpu.SemaphoreType.DMA,
            pltpu.SemaphoreType.DMA,
        ],
        compiler_params=pltpu.CompilerParams(use_tc_tiling_on_sc=False),
    )


T_BLK = 8192


def _repack_body(et_ref, o_ref):
    t = jnp.swapaxes(et_ref[...], 0, 1) * jnp.float32(1.0 / CTX)
    o_ref[:, :EMB] = t
    o_ref[:, EMB:] = jnp.zeros((T_BLK, EMB_PAD - EMB), jnp.float32)


def _repack(emb_t):
    return pl.pallas_call(
        _repack_body,
        grid=((VOCAB + T_BLK - 1) // T_BLK,),
        in_specs=[pl.BlockSpec((EMB, T_BLK), lambda i: (0, i))],
        out_specs=pl.BlockSpec((T_BLK, EMB_PAD), lambda i: (i, 0)),
        out_shape=jax.ShapeDtypeStruct((VOCAB, EMB_PAD), jnp.float32),
    )(emb_t)


V_BLK = 4096
_NVB = (VOCAB + V_BLK - 1) // V_BLK


def _mm_body(p_ref, wt_ref, b_ref, o_ref):
    p = p_ref[:, :EMB].astype(jnp.bfloat16)
    wt = wt_ref[...].astype(jnp.bfloat16)
    acc = lax.dot_general(
        wt, p, (((0,), (1,)), ((), ())), preferred_element_type=jnp.float32
    )
    o_ref[...] = acc + b_ref[...].T


def _project(pooled, W_T, b2d):
    out_t = pl.pallas_call(
        _mm_body,
        grid=(_NVB,),
        in_specs=[
            pl.BlockSpec((BATCH, EMB_PAD), lambda i: (0, 0)),
            pl.BlockSpec((EMB, V_BLK), lambda i: (0, i)),
            pl.BlockSpec((1, V_BLK), lambda i: (0, i)),
        ],
        out_specs=pl.BlockSpec((V_BLK, BATCH), lambda i: (i, 0)),
        out_shape=jax.ShapeDtypeStruct((VOCAB, BATCH), jnp.float32),
    )(pooled, W_T, b2d)
    return out_t.T


def kernel(inputs, emb_table, W, b):
    idx_flat = inputs.reshape(-1).astype(jnp.int32)
    table_s = _repack(emb_table.T)
    pooled = _make_pool_sc()(table_s, idx_flat)
    return _project(pooled, W.T, b.reshape(1, VOCAB))


# speedup vs baseline: 1.5087x; 1.0098x over previous
import functools

import jax
import jax.numpy as jnp
from jax import lax
from jax.experimental import pallas as pl
from jax.experimental.pallas import tpu as pltpu
from jax.experimental.pallas import tpu_sc as plsc

VOCAB = 100000
EMB = 64
EMB_PAD = 128
BATCH = 1024
CTX = 50

NC = 2
NS = 16
LANES = 16
NW = NC * NS
ITEMS_PER_W = BATCH // NW
ROWS_PER_W = ITEMS_PER_W * CTX
NPASS = 4
ITEMS_PER_PASS = ITEMS_PER_W // NPASS
ROWS_PER_PASS = ITEMS_PER_PASS * CTX
GCHUNK = 80
NSTREAM = ROWS_PER_PASS // GCHUNK


def _pool_sc_body(table_hbm, idx_hbm, out_hbm, idx_v, rows0, rows1, acc_v,
                  sem0, sem1):
    wid = lax.axis_index("s") * NC + lax.axis_index("c")
    base = wid * ROWS_PER_W
    pltpu.sync_copy(idx_hbm.at[pl.ds(base, ROWS_PER_W)], idx_v)

    def fire(p, buf, sem):
        for s in range(NSTREAM):
            pltpu.make_async_copy(
                table_hbm.at[idx_v.at[pl.ds(p * ROWS_PER_PASS + s * GCHUNK,
                                            GCHUNK)]],
                buf.at[pl.ds(s * GCHUNK, GCHUNK)],
                sem,
            ).start()

    def drain(p, buf, sem):
        for s in range(NSTREAM):
            pltpu.make_async_copy(
                table_hbm.at[idx_v.at[pl.ds(p * ROWS_PER_PASS + s * GCHUNK,
                                            GCHUNK)]],
                buf.at[pl.ds(s * GCHUNK, GCHUNK)],
                sem,
            ).wait()

    def accum(p, buf):
        @pl.loop(0, ITEMS_PER_PASS)
        def _(j):
            r0 = j * CTX

            def body(r, accs):
                return tuple(
                    accs[c] + buf[r0 + r, pl.ds(c * LANES, LANES)]
                    for c in range(EMB // LANES)
                )

            init = tuple(
                buf[r0, pl.ds(c * LANES, LANES)] for c in range(EMB // LANES)
            )
            accs = lax.fori_loop(1, CTX, body, init)
            it = p * ITEMS_PER_PASS + j
            for c in range(EMB // LANES):
                acc_v[it, pl.ds(c * LANES, LANES)] = accs[c]

    bufs = (rows0, rows1)
    sems = (sem0, sem1)
    fire(0, rows0, sem0)
    fire(1, rows1, sem1)
    for p in range(NPASS):
        b = p % 2
        drain(p, bufs[b], sems[b])
        accum(p, bufs[b])
        if p + 2 < NPASS:
            fire(p + 2, bufs[b], sems[b])
    pltpu.sync_copy(acc_v, out_hbm.at[pl.ds(wid * ITEMS_PER_W, ITEMS_PER_W)])


@functools.cache
def _make_pool_sc():
    mesh = plsc.VectorSubcoreMesh(core_axis_name="c", subcore_axis_name="s")
    return pl.kernel(
        _pool_sc_body,
        mesh=mesh,
        out_type=jax.ShapeDtypeStruct((BATCH, EMB_PAD), jnp.float32),
        scratch_types=[
            pltpu.VMEM((ROWS_PER_W,), jnp.int32),
            pltpu.VMEM((ROWS_PER_PASS, EMB_PAD), jnp.float32),
            pltpu.VMEM((ROWS_PER_PASS, EMB_PAD), jnp.float32),
            pltpu.VMEM((ITEMS_PER_W, EMB_PAD), jnp.float32),
            pltpu.SemaphoreType.DMA,
            pltpu.SemaphoreType.DMA,
        ],
        compiler_params=pltpu.CompilerParams(use_tc_tiling_on_sc=False),
    )


T_BLK = 16384


def _repack_body(et_ref, o_ref):
    t = jnp.swapaxes(et_ref[...], 0, 1) * jnp.float32(1.0 / CTX)
    o_ref[:, :EMB] = t
    o_ref[:, EMB:] = jnp.zeros((T_BLK, EMB_PAD - EMB), jnp.float32)


def _repack(emb_t):
    return pl.pallas_call(
        _repack_body,
        grid=((VOCAB + T_BLK - 1) // T_BLK,),
        in_specs=[pl.BlockSpec((EMB, T_BLK), lambda i: (0, i))],
        out_specs=pl.BlockSpec((T_BLK, EMB_PAD), lambda i: (i, 0)),
        out_shape=jax.ShapeDtypeStruct((VOCAB, EMB_PAD), jnp.float32),
    )(emb_t)


V_BLK = 4096
_NVB = (VOCAB + V_BLK - 1) // V_BLK


def _mm_body(p_ref, wt_ref, b_ref, o_ref):
    p = p_ref[:, :EMB].astype(jnp.bfloat16)
    wt = wt_ref[...].astype(jnp.bfloat16)
    acc = lax.dot_general(
        wt, p, (((0,), (1,)), ((), ())), preferred_element_type=jnp.float32
    )
    o_ref[...] = acc + b_ref[...].T


def _project(pooled, W_T, b2d):
    out_t = pl.pallas_call(
        _mm_body,
        grid=(_NVB,),
        in_specs=[
            pl.BlockSpec((BATCH, EMB_PAD), lambda i: (0, 0)),
            pl.BlockSpec((EMB, V_BLK), lambda i: (0, i)),
            pl.BlockSpec((1, V_BLK), lambda i: (0, i)),
        ],
        out_specs=pl.BlockSpec((V_BLK, BATCH), lambda i: (i, 0)),
        out_shape=jax.ShapeDtypeStruct((VOCAB, BATCH), jnp.float32),
    )(pooled, W_T, b2d)
    return out_t.T


def kernel(inputs, emb_table, W, b):
    idx_flat = inputs.reshape(-1).astype(jnp.int32)
    table_s = _repack(emb_table.T)
    pooled = _make_pool_sc()(table_s, idx_flat)
    return _project(pooled, W.T, b.reshape(1, VOCAB))


# speedup vs baseline: 1.5087x; 1.0000x over previous
import functools

import jax
import jax.numpy as jnp
from jax import lax
from jax.experimental import pallas as pl
from jax.experimental.pallas import tpu as pltpu
from jax.experimental.pallas import tpu_sc as plsc

VOCAB = 100000
EMB = 64
EMB_PAD = 128
BATCH = 1024
CTX = 50

NC = 2
NS = 16
LANES = 16
NW = NC * NS
ITEMS_PER_W = BATCH // NW
ROWS_PER_W = ITEMS_PER_W * CTX
NPASS = 4
ITEMS_PER_PASS = ITEMS_PER_W // NPASS
ROWS_PER_PASS = ITEMS_PER_PASS * CTX
GCHUNK = 80
NSTREAM = ROWS_PER_PASS // GCHUNK


def _pool_sc_body(table_hbm, idx_hbm, out_hbm, idx_v, rows0, rows1, acc_v,
                  sem0, sem1):
    wid = lax.axis_index("s") * NC + lax.axis_index("c")
    base = wid * ROWS_PER_W
    pltpu.sync_copy(idx_hbm.at[pl.ds(base, ROWS_PER_W)], idx_v)

    def fire(p, buf, sem):
        for s in range(NSTREAM):
            pltpu.make_async_copy(
                table_hbm.at[idx_v.at[pl.ds(p * ROWS_PER_PASS + s * GCHUNK,
                                            GCHUNK)]],
                buf.at[pl.ds(s * GCHUNK, GCHUNK)],
                sem,
            ).start()

    def drain(p, buf, sem):
        for s in range(NSTREAM):
            pltpu.make_async_copy(
                table_hbm.at[idx_v.at[pl.ds(p * ROWS_PER_PASS + s * GCHUNK,
                                            GCHUNK)]],
                buf.at[pl.ds(s * GCHUNK, GCHUNK)],
                sem,
            ).wait()

    def accum(p, buf):
        @pl.loop(0, ITEMS_PER_PASS)
        def _(j):
            r0 = j * CTX

            def body(r, accs):
                return tuple(
                    accs[c] + buf[r0 + r, pl.ds(c * LANES, LANES)]
                    for c in range(EMB // LANES)
                )

            init = tuple(
                buf[r0, pl.ds(c * LANES, LANES)] for c in range(EMB // LANES)
            )
            accs = lax.fori_loop(1, CTX, body, init)
            it = p * ITEMS_PER_PASS + j
            for c in range(EMB // LANES):
                acc_v[it, pl.ds(c * LANES, LANES)] = accs[c]

    bufs = (rows0, rows1)
    sems = (sem0, sem1)
    fire(0, rows0, sem0)
    fire(1, rows1, sem1)
    for p in range(NPASS):
        b = p % 2
        drain(p, bufs[b], sems[b])
        accum(p, bufs[b])
        if p + 2 < NPASS:
            fire(p + 2, bufs[b], sems[b])
    pltpu.sync_copy(acc_v, out_hbm.at[pl.ds(wid * ITEMS_PER_W, ITEMS_PER_W)])


@functools.cache
def _make_pool_sc():
    mesh = plsc.VectorSubcoreMesh(core_axis_name="c", subcore_axis_name="s")
    return pl.kernel(
        _pool_sc_body,
        mesh=mesh,
        out_type=jax.ShapeDtypeStruct((BATCH, EMB_PAD), jnp.float32),
        scratch_types=[
            pltpu.VMEM((ROWS_PER_W,), jnp.int32),
            pltpu.VMEM((ROWS_PER_PASS, EMB_PAD), jnp.float32),
            pltpu.VMEM((ROWS_PER_PASS, EMB_PAD), jnp.float32),
            pltpu.VMEM((ITEMS_PER_W, EMB_PAD), jnp.float32),
            pltpu.SemaphoreType.DMA,
            pltpu.SemaphoreType.DMA,
        ],
        compiler_params=pltpu.CompilerParams(use_tc_tiling_on_sc=False),
    )


T_BLK = 16384


def _repack_body(et_ref, o_ref):
    t = jnp.swapaxes(et_ref[...], 0, 1) * jnp.float32(1.0 / CTX)
    o_ref[:, :EMB] = t
    o_ref[:, EMB:] = jnp.zeros((T_BLK, EMB_PAD - EMB), jnp.float32)


def _repack(emb_t):
    return pl.pallas_call(
        _repack_body,
        grid=((VOCAB + T_BLK - 1) // T_BLK,),
        in_specs=[pl.BlockSpec((EMB, T_BLK), lambda i: (0, i))],
        out_specs=pl.BlockSpec((T_BLK, EMB_PAD), lambda i: (i, 0)),
        out_shape=jax.ShapeDtypeStruct((VOCAB, EMB_PAD), jnp.float32),
    )(emb_t)


V_BLK = 5120
_NVB = (VOCAB + V_BLK - 1) // V_BLK


def _mm_body(p_ref, wt_ref, b_ref, o_ref):
    p = p_ref[:, :EMB].astype(jnp.bfloat16)
    wt = wt_ref[...].astype(jnp.bfloat16)
    acc = lax.dot_general(
        wt, p, (((0,), (1,)), ((), ())), preferred_element_type=jnp.float32
    )
    o_ref[...] = acc + b_ref[...].T


def _project(pooled, W_T, b2d):
    out_t = pl.pallas_call(
        _mm_body,
        grid=(_NVB,),
        in_specs=[
            pl.BlockSpec((BATCH, EMB_PAD), lambda i: (0, 0)),
            pl.BlockSpec((EMB, V_BLK), lambda i: (0, i)),
            pl.BlockSpec((1, V_BLK), lambda i: (0, i)),
        ],
        out_specs=pl.BlockSpec((V_BLK, BATCH), lambda i: (i, 0)),
        out_shape=jax.ShapeDtypeStruct((VOCAB, BATCH), jnp.float32),
    )(pooled, W_T, b2d)
    return out_t.T


def kernel(inputs, emb_table, W, b):
    idx_flat = inputs.reshape(-1).astype(jnp.int32)
    table_s = _repack(emb_table.T)
    pooled = _make_pool_sc()(table_s, idx_flat)
    return _project(pooled, W.T, b.reshape(1, VOCAB))
